# Initial kernel scaffold; baseline (speedup 1.0000x reference)
import jax
import jax.numpy as jnp
from jax.experimental import pallas as pl


def kernel(X, table):
    raise NotImplementedError("write your pallas kernel here")


# speedup vs baseline: 1.9668x; 1.9668x over previous
import functools

import jax
import jax.numpy as jnp
from jax import lax
from jax.experimental import pallas as pl
from jax.experimental.pallas import tpu as pltpu
from jax.experimental.pallas import tpu_sc as plsc

_VOCAB = 8192
_B, _T = 16, 512
_N = _B * _T
_NC, _NS = 2, 16
_NW = _NC * _NS
_RPW = _N // _NW
_CH = 4
_NBUF = 2
_NCH = _RPW // _CH

_mesh = plsc.VectorSubcoreMesh(core_axis_name="c", subcore_axis_name="s")


@functools.partial(
    pl.kernel,
    mesh=_mesh,
    out_type=jax.ShapeDtypeStruct((_N, _VOCAB), jnp.float32),
    scratch_types=[
        pltpu.VMEM((_NCH, _CH), jnp.int32),
        pltpu.VMEM((_CH, _VOCAB), jnp.float32),
        pltpu.VMEM((_CH, _VOCAB), jnp.float32),
        pltpu.SemaphoreType.DMA,
        pltpu.SemaphoreType.DMA,
        pltpu.SemaphoreType.DMA,
        pltpu.SemaphoreType.DMA,
    ],
)
def _gather_rows(x_hbm, table_hbm, out_hbm, idx_v, buf0, buf1, g0, g1, s0, s1):
    wid = lax.axis_index("s") * _NC + lax.axis_index("c")
    base = wid * _RPW
    bufs = (buf0, buf1)
    gsems = (g0, g1)
    ssems = (s0, s1)

    pltpu.sync_copy(x_hbm.at[wid], idx_v)

    def gdesc(chunk, b):
        return pltpu.make_async_copy(
            table_hbm.at[idx_v.at[chunk]], bufs[b], gsems[b])

    def sdesc(chunk, b):
        return pltpu.make_async_copy(
            bufs[b], out_hbm.at[pl.ds(base + chunk * _CH, _CH)], ssems[b])

    for b in range(_NBUF):
        gdesc(b, b).start()

    def body(o, carry):
        for b in range(_NBUF):
            chunk = o * _NBUF + b
            gdesc(chunk, b).wait()
            sdesc(chunk, b).start()
            sdesc(chunk, b).wait()
            gdesc(chunk + _NBUF, b).start()
        return carry

    lax.fori_loop(0, _NCH // _NBUF - 1, body, 0)

    for b in range(_NBUF):
        chunk = _NCH - _NBUF + b
        gdesc(chunk, b).wait()
        sdesc(chunk, b).start()
    for b in range(_NBUF):
        sdesc(_NCH - _NBUF + b, b).wait()


def kernel(X, table):
    xf = X.reshape(_NW, _NCH, _CH).astype(jnp.int32)
    out = _gather_rows(xf, table)
    return out.reshape(_B, _T, _VOCAB)
